# baseline (speedup 1.0000x reference)
import functools

import jax
import jax.numpy as jnp
import numpy as np
from jax import lax
from jax.experimental import pallas as pl
from jax.experimental.pallas import tpu as pltpu
from jax.experimental.pallas import tpu_sc as plsc

_SIGMA_I = 10.0
_SIGMA_X = 4.0
_R = 5
_SAMPLE_NUM = 1000
_H = 224
_W = 224
_N = 4
_K = 21
_C = 3
_SPAD = 1024
_PIX = _H * _W

_offs = []
_expos = []
for _i in range(-_R, _R + 1):
    for _j in range(-_R, _R + 1):
        if _i * _i + _j * _j <= _R * _R:
            _offs.append(_i * _W + _j)
            _expos.append(float(np.exp(-(_i * _i + _j * _j) / _SIGMA_X ** 2)))
_P = len(_offs)

_rng = np.random.default_rng(0)
_h_s = _rng.integers(_R, _H - _R, _SAMPLE_NUM)
_w_s = _rng.integers(_R, _W - _R, _SAMPLE_NUM)
_cidx_np = np.zeros(_SPAD, np.int32)
_cidx_np[:_SAMPLE_NUM] = (_h_s * _W + _w_s).astype(np.int32)
_cidx_np[_SAMPLE_NUM:] = _cidx_np[0]

_NC = 2
_NS = 16
_NW = _NC * _NS
_LANES = 16

_N_PER_SC = _N // _NC
_A_CHUNKS = _NS // _N_PER_SC
_A_CSZ = _SPAD // _A_CHUNKS
_NPLANES = _N * _K
_SC_PLANES = _N_PER_SC * _K
_B_HSZ = 256

_mesh = plsc.VectorSubcoreMesh(
    core_axis_name="c", subcore_axis_name="s", num_cores=_NC, num_subcores=_NS
)


def _fused_body(imgs_hbm, pred_hbm, cidx_hbm, num_hbm, den_hbm, w_hbm,
                wsum_hbm, plane_v, acc2_v, wb_v, cidx_v, wsumb_v, wsuma_v,
                out_v):
    cid = lax.axis_index("c")
    sid = lax.axis_index("s")

    pltpu.sync_copy(cidx_hbm, cidx_v)

    neg_inv_s2 = -1.0 / (_SIGMA_I ** 2)

    n_local = sid // _A_CHUNKS
    chunk = sid % _A_CHUNKS
    base = chunk * _A_CSZ
    n = _N_PER_SC * cid + n_local

    for c in range(_C):
        pltpu.sync_copy(imgs_hbm.at[n * _C + c], plane_v)

        def sv_body(sv, _, c=c):
            cvec = cidx_v[pl.ds(base + sv * _LANES, _LANES)]
            centv = plsc.load_gather(plane_v, [cvec])
            for p in range(_P):
                gv = plsc.load_gather(plane_v, [cvec + _offs[p]])
                d = gv - centv
                if c == 0:
                    w_slc = acc2_v.at[p, pl.ds(sv * _LANES, _LANES)]
                    w_slc[...] = d * d
                else:
                    plsc.addupdate(acc2_v.at[p, pl.ds(sv * _LANES, _LANES)], d * d)
            return _

        lax.fori_loop(0, _A_CSZ // _LANES, sv_body, None)

    def exp_body(sv, _):
        s_base = base + sv * _LANES
        lanes = lax.iota(jnp.int32, _LANES) + s_base
        valid = lanes < _SAMPLE_NUM
        wsum = jnp.zeros((_LANES,), jnp.float32)
        for p in range(_P):
            a = acc2_v[p, pl.ds(sv * _LANES, _LANES)]
            wv = jnp.exp(a * neg_inv_s2) * _expos[p]
            wv = jnp.where(valid, wv, 0.0)
            acc2_v[p, pl.ds(sv * _LANES, _LANES)] = wv
            wsum = wsum + wv
        wsuma_v[pl.ds(sv * _LANES, _LANES)] = wsum
        return _

    lax.fori_loop(0, _A_CSZ // _LANES, exp_body, None)

    pltpu.sync_copy(acc2_v, w_hbm.at[n, :, pl.ds(base, _A_CSZ)])
    pltpu.sync_copy(wsuma_v, wsum_hbm.at[n, pl.ds(base, _A_CSZ)])

    plsc.subcore_barrier()

    n_slots = (_SC_PLANES + _NS - 1) // _NS

    def slot_body(slot, _):
        jl = sid + slot * _NS

        @pl.when(jl < _SC_PLANES)
        def _do_job():
            nl = jl // _K
            j = _SC_PLANES * cid + jl
            nb = _N_PER_SC * cid + nl
            pltpu.sync_copy(pred_hbm.at[j], plane_v)
            pltpu.sync_copy(wsum_hbm.at[nb], wsumb_v)

            def half_body(half, carry):
                num_h, den_h = carry
                pltpu.sync_copy(w_hbm.at[nb, :, pl.ds(half * _B_HSZ, _B_HSZ)],
                                wb_v)

                def sv_body(sv, carry2):
                    num_c, den_c = carry2
                    s_off = half * _B_HSZ + sv * _LANES
                    cvec = cidx_v[pl.ds(s_off, _LANES)]
                    pv = plsc.load_gather(plane_v, [cvec])
                    qw = jnp.zeros((_LANES,), jnp.float32)
                    for p in range(_P):
                        qv = plsc.load_gather(plane_v, [cvec + _offs[p]])
                        wv = wb_v[p, pl.ds(sv * _LANES, _LANES)]
                        qw = qw + wv * qv
                    wsv = wsumb_v[pl.ds(s_off, _LANES)]
                    return num_c + qw * pv, den_c + wsv * pv

                return lax.fori_loop(0, _B_HSZ // _LANES, sv_body,
                                     (num_h, den_h))

            num_acc, den_acc = lax.fori_loop(
                0, _SPAD // _B_HSZ, half_body,
                (jnp.zeros((_LANES,), jnp.float32),
                 jnp.zeros((_LANES,), jnp.float32)))

            out_v[...] = num_acc
            pltpu.sync_copy(out_v, num_hbm.at[j])
            out_v[...] = den_acc
            pltpu.sync_copy(out_v, den_hbm.at[j])

        return _

    lax.fori_loop(0, n_slots, slot_body, None)


def _loss_body(num_ref, den_ref, out_ref):
    num = jnp.sum(num_ref[...], axis=1)
    den = jnp.sum(den_ref[...], axis=1)
    out_ref[...] = jnp.reshape(_N * _K - jnp.sum(num / den), (1, 1))


@jax.jit
def kernel(predictions, imgs):
    pred_flat = predictions.reshape(_NPLANES, _PIX)
    imgs_flat = imgs.reshape(_N * _C, _PIX)
    cidx = jnp.asarray(_cidx_np)

    fused_k = pl.kernel(
        _fused_body,
        out_type=(
            jax.ShapeDtypeStruct((_NPLANES, _LANES), jnp.float32),
            jax.ShapeDtypeStruct((_NPLANES, _LANES), jnp.float32),
            jax.ShapeDtypeStruct((_N, _P, _SPAD), jnp.float32),
            jax.ShapeDtypeStruct((_N, _SPAD), jnp.float32),
        ),
        mesh=_mesh,
        scratch_types=[
            pltpu.VMEM((_PIX,), jnp.float32),
            pltpu.VMEM((_P, _A_CSZ), jnp.float32),
            pltpu.VMEM((_P, _B_HSZ), jnp.float32),
            pltpu.VMEM((_SPAD,), jnp.int32),
            pltpu.VMEM((_SPAD,), jnp.float32),
            pltpu.VMEM((_A_CSZ,), jnp.float32),
            pltpu.VMEM((_LANES,), jnp.float32),
        ],
        compiler_params=pltpu.CompilerParams(needs_layout_passes=False),
    )
    num_p, den_p, _w_unused, _ws_unused = fused_k(imgs_flat, pred_flat, cidx)

    loss = pl.pallas_call(
        _loss_body,
        out_shape=jax.ShapeDtypeStruct((1, 1), jnp.float32),
    )(num_p, den_p)
    return loss[0, 0]


# speedup vs baseline: 1.0530x; 1.0530x over previous
import functools

import jax
import jax.numpy as jnp
import numpy as np
from jax import lax
from jax.experimental import pallas as pl
from jax.experimental.pallas import tpu as pltpu
from jax.experimental.pallas import tpu_sc as plsc

_SIGMA_I = 10.0
_SIGMA_X = 4.0
_R = 5
_SAMPLE_NUM = 1000
_H = 224
_W = 224
_N = 4
_K = 21
_C = 3
_SPAD = 1024
_PIX = _H * _W

_offs = []
_expos = []
for _i in range(-_R, _R + 1):
    for _j in range(-_R, _R + 1):
        if _i * _i + _j * _j <= _R * _R:
            _offs.append(_i * _W + _j)
            _expos.append(float(np.exp(-(_i * _i + _j * _j) / _SIGMA_X ** 2)))
_P = len(_offs)

_rng = np.random.default_rng(0)
_h_s = _rng.integers(_R, _H - _R, _SAMPLE_NUM)
_w_s = _rng.integers(_R, _W - _R, _SAMPLE_NUM)
_cidx_np = np.zeros(_SPAD, np.int32)
_cidx_np[:_SAMPLE_NUM] = (_h_s * _W + _w_s).astype(np.int32)
_cidx_np[_SAMPLE_NUM:] = _cidx_np[0]

_NC = 2
_NS = 16
_NW = _NC * _NS
_LANES = 16
_CHUNK = _SPAD // _NW

_A_CHUNKS = 8
_A_CSZ = _SPAD // _A_CHUNKS
_NPLANES = _N * _K

_mesh = plsc.VectorSubcoreMesh(
    core_axis_name="c", subcore_axis_name="s", num_cores=_NC, num_subcores=_NS
)


def _weights_body(imgs_hbm, cidx_hbm, w_hbm, wsum_hbm, plane0_v, plane1_v,
                  acc2_v, w_v, wsum_v, cidx_v, sem0, sem1):
    wid = lax.axis_index("s") * _NC + lax.axis_index("c")
    n = wid // _A_CHUNKS
    chunk = wid % _A_CHUNKS
    base = chunk * _A_CSZ

    pltpu.sync_copy(cidx_hbm.at[pl.ds(base, _A_CSZ)], cidx_v)

    neg_inv_s2 = -1.0 / (_SIGMA_I ** 2)
    planes = [plane0_v, plane1_v]
    sems = [sem0, sem1]

    copies = {
        0: pltpu.async_copy(imgs_hbm.at[n * _C + 0], planes[0], sems[0]),
        1: pltpu.async_copy(imgs_hbm.at[n * _C + 1], planes[1], sems[1]),
    }
    for c in range(_C):
        buf = planes[c % 2]
        with jax.named_scope(f"A_wait_ch{c}"):
            copies[c].wait()

        def sv_body(sv, _, c=c, buf=buf):
            cvec = cidx_v[pl.ds(sv * _LANES, _LANES)]
            centv = plsc.load_gather(buf, [cvec])
            for p in range(_P):
                gv = plsc.load_gather(buf, [cvec + _offs[p]])
                d = gv - centv
                if c == 0:
                    w_slc = acc2_v.at[p, pl.ds(sv * _LANES, _LANES)]
                    w_slc[...] = d * d
                else:
                    plsc.addupdate(acc2_v.at[p, pl.ds(sv * _LANES, _LANES)], d * d)
            return _

        with jax.named_scope(f"A_gather_ch{c}"):
            lax.fori_loop(0, _A_CSZ // _LANES, sv_body, None)
        if c + 2 < _C:
            copies[c + 2] = pltpu.async_copy(
                imgs_hbm.at[n * _C + c + 2], planes[c % 2], sems[c % 2])

    def exp_body(sv, _):
        s_base = base + sv * _LANES
        lanes = lax.iota(jnp.int32, _LANES) + s_base
        valid = lanes < _SAMPLE_NUM
        wsums = [jnp.zeros((_LANES,), jnp.float32) for _i in range(3)]
        for p in range(_P):
            a = acc2_v[p, pl.ds(sv * _LANES, _LANES)]
            wv = jnp.exp(a * neg_inv_s2) * _expos[p]
            wv = jnp.where(valid, wv, 0.0)
            w_v[p, pl.ds(sv * _LANES, _LANES)] = wv
            wsums[p % 3] = wsums[p % 3] + wv
        wsum_v[pl.ds(sv * _LANES, _LANES)] = wsums[0] + (wsums[1] + wsums[2])
        return _

    with jax.named_scope("A_exp"):
        lax.fori_loop(0, _A_CSZ // _LANES, exp_body, None)

    with jax.named_scope("A_writeout"):
        pltpu.sync_copy(w_v, w_hbm.at[n, :, pl.ds(base, _A_CSZ)])
        pltpu.sync_copy(wsum_v, wsum_hbm.at[n, pl.ds(base, _A_CSZ)])


_B_QSZ = 512


def _numden_body(pred_hbm, w_all_hbm, wsum_hbm, cidx_hbm, num_hbm, den_hbm,
                 plane_v, w0_v, wsum_v, cidx_v, out_v, sem_p, sem_w0,
                 sem_w1):
    wid = lax.axis_index("s") * _NC + lax.axis_index("c")

    pltpu.sync_copy(cidx_hbm, cidx_v)
    w_bufs = [w0_v, w0_v]
    n_q = _SPAD // _B_QSZ

    n_slots = (_NPLANES + _NW - 1) // _NW
    for slot in range(n_slots):
        j = wid + slot * _NW

        def do_job(j=j):
            n = j // _K
            with jax.named_scope("B_plane_dma"):
                pltpu.sync_copy(pred_hbm.at[j], plane_v)
                pltpu.sync_copy(wsum_hbm.at[n], wsum_v)

            num_acc = jnp.zeros((_LANES,), jnp.float32)
            den_acc = jnp.zeros((_LANES,), jnp.float32)
            for q in range(n_q):
                w_buf = w_bufs[q % 2]
                with jax.named_scope("B_w_dma"):
                    pltpu.sync_copy(
                        w_all_hbm.at[n, :, pl.ds(q * _B_QSZ, _B_QSZ)], w_buf)

                def sv_body(sv, carry, q=q, w_buf=w_buf):
                    num_c, den_c = carry
                    s_off = q * _B_QSZ + sv * _LANES
                    cvec = cidx_v[pl.ds(s_off, _LANES)]
                    pv = plsc.load_gather(plane_v, [cvec])
                    qw = jnp.zeros((_LANES,), jnp.float32)
                    for p in range(_P):
                        qv = plsc.load_gather(plane_v, [cvec + _offs[p]])
                        wv = w_buf[p, pl.ds(sv * _LANES, _LANES)]
                        qw = qw + wv * qv
                    wsv = wsum_v[pl.ds(s_off, _LANES)]
                    return num_c + qw * pv, den_c + wsv * pv

                with jax.named_scope("B_compute"):
                    num_acc, den_acc = lax.fori_loop(
                        0, _B_QSZ // _LANES, sv_body, (num_acc, den_acc))

            out_v[...] = num_acc
            pltpu.sync_copy(out_v, num_hbm.at[j])
            out_v[...] = den_acc
            pltpu.sync_copy(out_v, den_hbm.at[j])

        if (slot + 1) * _NW <= _NPLANES:
            do_job()
        else:
            @pl.when(j < _NPLANES)
            def _():
                do_job()


def _loss_body(num_ref, den_ref, out_ref):
    num = jnp.sum(num_ref[...], axis=1)
    den = jnp.sum(den_ref[...], axis=1)
    out_ref[...] = jnp.reshape(_N * _K - jnp.sum(num / den), (1, 1))


@jax.jit
def kernel(predictions, imgs):
    pred_flat = predictions.reshape(_NPLANES, _PIX)
    imgs_flat = imgs.reshape(_N * _C, _PIX)
    cidx = jnp.asarray(_cidx_np)

    weights_k = pl.kernel(
        _weights_body,
        out_type=(
            jax.ShapeDtypeStruct((_N, _P, _SPAD), jnp.float32),
            jax.ShapeDtypeStruct((_N, _SPAD), jnp.float32),
        ),
        mesh=_mesh,
        scratch_types=[
            pltpu.VMEM((_PIX,), jnp.float32),
            pltpu.VMEM((_PIX,), jnp.float32),
            pltpu.VMEM((_P, _A_CSZ), jnp.float32),
            pltpu.VMEM((_P, _A_CSZ), jnp.float32),
            pltpu.VMEM((_A_CSZ,), jnp.float32),
            pltpu.VMEM((_A_CSZ,), jnp.int32),
            pltpu.SemaphoreType.DMA,
            pltpu.SemaphoreType.DMA,
        ],
        compiler_params=pltpu.CompilerParams(needs_layout_passes=False),
    )
    w_all, wsum = weights_k(imgs_flat, cidx)

    numden_k = pl.kernel(
        _numden_body,
        out_type=(
            jax.ShapeDtypeStruct((_NPLANES, _LANES), jnp.float32),
            jax.ShapeDtypeStruct((_NPLANES, _LANES), jnp.float32),
        ),
        mesh=_mesh,
        scratch_types=[
            pltpu.VMEM((_PIX,), jnp.float32),
            pltpu.VMEM((_P, _B_QSZ), jnp.float32),
            pltpu.VMEM((_SPAD,), jnp.float32),
            pltpu.VMEM((_SPAD,), jnp.int32),
            pltpu.VMEM((_LANES,), jnp.float32),
            pltpu.SemaphoreType.DMA,
            pltpu.SemaphoreType.DMA,
            pltpu.SemaphoreType.DMA,
        ],
        compiler_params=pltpu.CompilerParams(needs_layout_passes=False),
    )
    num_p, den_p = numden_k(pred_flat, w_all, wsum, cidx)

    loss = pl.pallas_call(
        _loss_body,
        out_shape=jax.ShapeDtypeStruct((1, 1), jnp.float32),
    )(num_p, den_p)
    return loss[0, 0]


# speedup vs baseline: 1.0821x; 1.0277x over previous
import functools

import jax
import jax.numpy as jnp
import numpy as np
from jax import lax
from jax.experimental import pallas as pl
from jax.experimental.pallas import tpu as pltpu
from jax.experimental.pallas import tpu_sc as plsc

_SIGMA_I = 10.0
_SIGMA_X = 4.0
_R = 5
_SAMPLE_NUM = 1000
_H = 224
_W = 224
_N = 4
_K = 21
_C = 3
_SPAD = 1024
_PIX = _H * _W

_offs = []
_expos = []
for _i in range(-_R, _R + 1):
    for _j in range(-_R, _R + 1):
        if _i * _i + _j * _j <= _R * _R:
            _offs.append(_i * _W + _j)
            _expos.append(float(np.exp(-(_i * _i + _j * _j) / _SIGMA_X ** 2)))
_P = len(_offs)

_rng = np.random.default_rng(0)
_h_s = _rng.integers(_R, _H - _R, _SAMPLE_NUM)
_w_s = _rng.integers(_R, _W - _R, _SAMPLE_NUM)
_cidx_np = np.zeros(_SPAD, np.int32)
_cidx_np[:_SAMPLE_NUM] = (_h_s * _W + _w_s).astype(np.int32)
_cidx_np[_SAMPLE_NUM:] = _cidx_np[0]

_NC = 2
_NS = 16
_NW = _NC * _NS
_LANES = 16
_CHUNK = _SPAD // _NW

_A_CHUNKS = 8
_A_CSZ = _SPAD // _A_CHUNKS
_NPLANES = _N * _K

_mesh = plsc.VectorSubcoreMesh(
    core_axis_name="c", subcore_axis_name="s", num_cores=_NC, num_subcores=_NS
)


def _weights_body(imgs_hbm, cidx_hbm, w_hbm, wsum_hbm, plane0_v, plane1_v,
                  acc2_v, w_v, wsum_v, cidx_v, sem0, sem1):
    wid = lax.axis_index("s") * _NC + lax.axis_index("c")
    n = wid // _A_CHUNKS
    chunk = wid % _A_CHUNKS
    base = chunk * _A_CSZ

    pltpu.sync_copy(cidx_hbm.at[pl.ds(base, _A_CSZ)], cidx_v)

    neg_inv_s2 = -1.0 / (_SIGMA_I ** 2)
    planes = [plane0_v, plane1_v]
    sems = [sem0, sem1]

    qsz = _PIX // 4

    def start_plane(c, bi):
        return [pltpu.async_copy(
            imgs_hbm.at[n * _C + c, pl.ds(i * qsz, qsz)],
            planes[bi].at[pl.ds(i * qsz, qsz)], sems[bi]) for i in range(4)]

    copies = {0: start_plane(0, 0), 1: start_plane(1, 1)}
    for c in range(_C):
        buf = planes[c % 2]
        with jax.named_scope(f"A_wait_ch{c}"):
            for _cp in copies[c]:
                _cp.wait()

        def sv_body(sv, _, c=c, buf=buf):
            cvec = cidx_v[pl.ds(sv * _LANES, _LANES)]
            centv = plsc.load_gather(buf, [cvec])
            for p in range(_P):
                gv = plsc.load_gather(buf, [cvec + _offs[p]])
                d = gv - centv
                if c == 0:
                    w_slc = acc2_v.at[p, pl.ds(sv * _LANES, _LANES)]
                    w_slc[...] = d * d
                else:
                    plsc.addupdate(acc2_v.at[p, pl.ds(sv * _LANES, _LANES)], d * d)
            return _

        with jax.named_scope(f"A_gather_ch{c}"):
            lax.fori_loop(0, _A_CSZ // _LANES, sv_body, None)
        if c + 2 < _C:
            copies[c + 2] = start_plane(c + 2, c % 2)

    def exp_body(sv, _):
        s_base = base + sv * _LANES
        lanes = lax.iota(jnp.int32, _LANES) + s_base
        valid = lanes < _SAMPLE_NUM
        wsums = [jnp.zeros((_LANES,), jnp.float32) for _i in range(3)]
        for p in range(_P):
            a = acc2_v[p, pl.ds(sv * _LANES, _LANES)]
            wv = jnp.exp(a * neg_inv_s2) * _expos[p]
            wv = jnp.where(valid, wv, 0.0)
            w_v[p, pl.ds(sv * _LANES, _LANES)] = wv
            wsums[p % 3] = wsums[p % 3] + wv
        wsum_v[pl.ds(sv * _LANES, _LANES)] = wsums[0] + (wsums[1] + wsums[2])
        return _

    with jax.named_scope("A_exp"):
        lax.fori_loop(0, _A_CSZ // _LANES, exp_body, None)

    with jax.named_scope("A_writeout"):
        pltpu.sync_copy(w_v, w_hbm.at[n, :, pl.ds(base, _A_CSZ)])
        pltpu.sync_copy(wsum_v, wsum_hbm.at[n, pl.ds(base, _A_CSZ)])


_B_QSZ = 256


def _numden_body(pred_hbm, w_all_hbm, wsum_hbm, cidx_hbm, num_hbm, den_hbm,
                 plane_v, w0_v, w1_v, wsum_v, cidx_v, out_v, sem_p, sem_w0,
                 sem_w1):
    wid = lax.axis_index("s") * _NC + lax.axis_index("c")

    pltpu.sync_copy(cidx_hbm, cidx_v)
    w_bufs = [w0_v, w1_v]
    w_sems = [sem_w0, sem_w1]
    n_q = _SPAD // _B_QSZ

    n_slots = (_NPLANES + _NW - 1) // _NW
    for slot in range(n_slots):
        j = wid + slot * _NW

        def do_job(j=j):
            n = j // _K
            psz = _PIX // 4
            plane_copies = [pltpu.async_copy(
                pred_hbm.at[j, pl.ds(i * psz, psz)],
                plane_v.at[pl.ds(i * psz, psz)], sem_p) for i in range(4)]
            w_copies = {0: pltpu.async_copy(
                w_all_hbm.at[n, :, pl.ds(0, _B_QSZ)], w_bufs[0], w_sems[0])}
            pltpu.sync_copy(wsum_hbm.at[n], wsum_v)
            with jax.named_scope("B_plane_dma"):
                for _cp in plane_copies:
                    _cp.wait()

            num_acc = jnp.zeros((_LANES,), jnp.float32)
            den_acc = jnp.zeros((_LANES,), jnp.float32)
            for q in range(n_q):
                w_buf = w_bufs[q % 2]
                with jax.named_scope("B_w_dma"):
                    w_copies[q].wait()
                if q + 1 < n_q:
                    w_copies[q + 1] = pltpu.async_copy(
                        w_all_hbm.at[n, :, pl.ds((q + 1) * _B_QSZ, _B_QSZ)],
                        w_bufs[(q + 1) % 2], w_sems[(q + 1) % 2])

                def sv_body(sv, carry, q=q, w_buf=w_buf):
                    num_c, den_c = carry
                    s_off = q * _B_QSZ + sv * _LANES
                    cvec = cidx_v[pl.ds(s_off, _LANES)]
                    pv = plsc.load_gather(plane_v, [cvec])
                    qw = jnp.zeros((_LANES,), jnp.float32)
                    for p in range(_P):
                        qv = plsc.load_gather(plane_v, [cvec + _offs[p]])
                        wv = w_buf[p, pl.ds(sv * _LANES, _LANES)]
                        qw = qw + wv * qv
                    wsv = wsum_v[pl.ds(s_off, _LANES)]
                    return num_c + qw * pv, den_c + wsv * pv

                with jax.named_scope("B_compute"):
                    num_acc, den_acc = lax.fori_loop(
                        0, _B_QSZ // _LANES, sv_body, (num_acc, den_acc))

            out_v[...] = num_acc
            pltpu.sync_copy(out_v, num_hbm.at[j])
            out_v[...] = den_acc
            pltpu.sync_copy(out_v, den_hbm.at[j])

        if (slot + 1) * _NW <= _NPLANES:
            do_job()
        else:
            @pl.when(j < _NPLANES)
            def _():
                do_job()


def _loss_body(num_ref, den_ref, out_ref):
    num = jnp.sum(num_ref[...], axis=1)
    den = jnp.sum(den_ref[...], axis=1)
    out_ref[...] = jnp.reshape(_N * _K - jnp.sum(num / den), (1, 1))


@jax.jit
def kernel(predictions, imgs):
    pred_flat = predictions.reshape(_NPLANES, _PIX)
    imgs_flat = imgs.reshape(_N * _C, _PIX)
    cidx = jnp.asarray(_cidx_np)

    weights_k = pl.kernel(
        _weights_body,
        out_type=(
            jax.ShapeDtypeStruct((_N, _P, _SPAD), jnp.float32),
            jax.ShapeDtypeStruct((_N, _SPAD), jnp.float32),
        ),
        mesh=_mesh,
        scratch_types=[
            pltpu.VMEM((_PIX,), jnp.float32),
            pltpu.VMEM((_PIX,), jnp.float32),
            pltpu.VMEM((_P, _A_CSZ), jnp.float32),
            pltpu.VMEM((_P, _A_CSZ), jnp.float32),
            pltpu.VMEM((_A_CSZ,), jnp.float32),
            pltpu.VMEM((_A_CSZ,), jnp.int32),
            pltpu.SemaphoreType.DMA,
            pltpu.SemaphoreType.DMA,
        ],
        compiler_params=pltpu.CompilerParams(needs_layout_passes=False),
    )
    w_all, wsum = weights_k(imgs_flat, cidx)

    numden_k = pl.kernel(
        _numden_body,
        out_type=(
            jax.ShapeDtypeStruct((_NPLANES, _LANES), jnp.float32),
            jax.ShapeDtypeStruct((_NPLANES, _LANES), jnp.float32),
        ),
        mesh=_mesh,
        scratch_types=[
            pltpu.VMEM((_PIX,), jnp.float32),
            pltpu.VMEM((_P, _B_QSZ), jnp.float32),
            pltpu.VMEM((_P, _B_QSZ), jnp.float32),
            pltpu.VMEM((_SPAD,), jnp.float32),
            pltpu.VMEM((_SPAD,), jnp.int32),
            pltpu.VMEM((_LANES,), jnp.float32),
            pltpu.SemaphoreType.DMA,
            pltpu.SemaphoreType.DMA,
            pltpu.SemaphoreType.DMA,
        ],
        compiler_params=pltpu.CompilerParams(needs_layout_passes=False),
    )
    num_p, den_p = numden_k(pred_flat, w_all, wsum, cidx)

    loss = pl.pallas_call(
        _loss_body,
        out_shape=jax.ShapeDtypeStruct((1, 1), jnp.float32),
    )(num_p, den_p)
    return loss[0, 0]


# speedup vs baseline: 1.1188x; 1.0338x over previous
import functools

import jax
import jax.numpy as jnp
import numpy as np
from jax import lax
from jax.experimental import pallas as pl
from jax.experimental.pallas import tpu as pltpu
from jax.experimental.pallas import tpu_sc as plsc

_SIGMA_I = 10.0
_SIGMA_X = 4.0
_R = 5
_SAMPLE_NUM = 1000
_H = 224
_W = 224
_N = 4
_K = 21
_C = 3
_SPAD = 1024
_PIX = _H * _W

_offs = []
_offs_ij = []
_expos = []
for _i in range(-_R, _R + 1):
    for _j in range(-_R, _R + 1):
        if _i * _i + _j * _j <= _R * _R:
            _offs.append(_i * _W + _j)
            _offs_ij.append((_i, _j))
            _expos.append(float(np.exp(-(_i * _i + _j * _j) / _SIGMA_X ** 2)))
_P = len(_offs)

_rng = np.random.default_rng(0)
_h_s = _rng.integers(_R, _H - _R, _SAMPLE_NUM)
_w_s = _rng.integers(_R, _W - _R, _SAMPLE_NUM)
_order = np.argsort(_h_s, kind="stable")
_h_sorted = _h_s[_order]
_w_sorted = _w_s[_order]
_cidx_np = np.zeros(_SPAD, np.int32)
_cidx_np[:_SAMPLE_NUM] = (_h_sorted * _W + _w_sorted).astype(np.int32)
_cidx_np[_SAMPLE_NUM:] = _cidx_np[_SAMPLE_NUM - 1]

_NC = 2
_NS = 16
_NW = _NC * _NS
_LANES = 16
_CHUNK = _SPAD // _NW

_A_CHUNKS = 8
_A_CSZ = _SPAD // _A_CHUNKS
_NPLANES = _N * _K

_h_pad = np.zeros(_SPAD, np.int64)
_h_pad[:_SAMPLE_NUM] = _h_sorted
_h_pad[_SAMPLE_NUM:] = _h_sorted[_SAMPLE_NUM - 1]
_start8 = []
_need_rows = []
for _k in range(_A_CHUNKS):
    _seg = _h_pad[_k * _A_CSZ:(_k + 1) * _A_CSZ]
    _s8 = (int(_seg.min()) - _R) & ~7
    _start8.append(_s8)
    _need_rows.append(int(_seg.max()) + _R - _s8 + 1)
_band_rows = -(-max(_need_rows) // 8) * 8
_BGROUPS = _band_rows // 8
_start8 = [min(_s, _H - _band_rows) for _s in _start8]
_h_pad_i = _h_pad.astype(np.int32)
_w_pad = np.zeros(_SPAD, np.int32)
_w_pad[:_SAMPLE_NUM] = _w_sorted
_w_pad[_SAMPLE_NUM:] = _w_sorted[_SAMPLE_NUM - 1]
_crow_np = np.zeros(_SPAD, np.int32)
for _k in range(_A_CHUNKS):
    _sl = slice(_k * _A_CSZ, (_k + 1) * _A_CSZ)
    _crow_np[_sl] = _h_pad_i[_sl] - _start8[_k]
_ccol_np = _w_pad

_mesh = plsc.VectorSubcoreMesh(
    core_axis_name="c", subcore_axis_name="s", num_cores=_NC, num_subcores=_NS
)


def _weights_body(imgs_hbm, crow_hbm, ccol_hbm, w_hbm, wsum_hbm, plane0_v,
                  plane1_v, acc2_v, w_v, wsum_v, crow_v, ccol_v, sem0, sem1):
    wid = lax.axis_index("s") * _NC + lax.axis_index("c")
    n = wid // _A_CHUNKS
    chunk = wid % _A_CHUNKS
    base = chunk * _A_CSZ

    pltpu.sync_copy(crow_hbm.at[pl.ds(base, _A_CSZ)], crow_v)
    pltpu.sync_copy(ccol_hbm.at[pl.ds(base, _A_CSZ)], ccol_v)

    neg_inv_s2 = -1.0 / (_SIGMA_I ** 2)
    planes = [plane0_v, plane1_v]
    sems = [sem0, sem1]

    bgrp = jnp.int32(_start8[0] // 8)
    for _k in range(1, _A_CHUNKS):
        bgrp = jnp.where(chunk == _k, jnp.int32(_start8[_k] // 8), bgrp)

    def start_plane(c, bi):
        return [pltpu.async_copy(
            imgs_hbm.at[n * _C + c, pl.ds(bgrp, _BGROUPS)],
            planes[bi], sems[bi])]

    copies = {0: start_plane(0, 0), 1: start_plane(1, 1)}
    for c in range(_C):
        buf = planes[c % 2]
        with jax.named_scope(f"A_wait_ch{c}"):
            for _cp in copies[c]:
                _cp.wait()

        def sv_body(sv, _, c=c, buf=buf):
            rvec = crow_v[pl.ds(sv * _LANES, _LANES)]
            kvec = ccol_v[pl.ds(sv * _LANES, _LANES)]
            centv = plsc.load_gather(buf, [rvec >> 3, rvec & 7, kvec])
            for p in range(_P):
                oi, oj = _offs_ij[p]
                rr = rvec + oi
                gv = plsc.load_gather(buf, [rr >> 3, rr & 7, kvec + oj])
                d = gv - centv
                if c == 0:
                    w_slc = acc2_v.at[p, pl.ds(sv * _LANES, _LANES)]
                    w_slc[...] = d * d
                else:
                    plsc.addupdate(acc2_v.at[p, pl.ds(sv * _LANES, _LANES)], d * d)
            return _

        with jax.named_scope(f"A_gather_ch{c}"):
            lax.fori_loop(0, _A_CSZ // _LANES, sv_body, None)
        if c + 2 < _C:
            copies[c + 2] = start_plane(c + 2, c % 2)

    def exp_body(sv, _):
        s_base = base + sv * _LANES
        lanes = lax.iota(jnp.int32, _LANES) + s_base
        valid = lanes < _SAMPLE_NUM
        wsums = [jnp.zeros((_LANES,), jnp.float32) for _i in range(3)]
        for p in range(_P):
            a = acc2_v[p, pl.ds(sv * _LANES, _LANES)]
            wv = jnp.exp(a * neg_inv_s2) * _expos[p]
            wv = jnp.where(valid, wv, 0.0)
            w_v[p, pl.ds(sv * _LANES, _LANES)] = wv
            wsums[p % 3] = wsums[p % 3] + wv
        wsum_v[pl.ds(sv * _LANES, _LANES)] = wsums[0] + (wsums[1] + wsums[2])
        return _

    with jax.named_scope("A_exp"):
        lax.fori_loop(0, _A_CSZ // _LANES, exp_body, None)

    with jax.named_scope("A_writeout"):
        pltpu.sync_copy(w_v, w_hbm.at[n, :, pl.ds(base, _A_CSZ)])
        pltpu.sync_copy(wsum_v, wsum_hbm.at[n, pl.ds(base, _A_CSZ)])


_B_QSZ = 256


def _numden_body(pred_hbm, w_all_hbm, wsum_hbm, cidx_hbm, num_hbm, den_hbm,
                 plane_v, w0_v, w1_v, wsum_v, cidx_v, out_v, sem_p, sem_w0,
                 sem_w1):
    wid = lax.axis_index("s") * _NC + lax.axis_index("c")

    pltpu.sync_copy(cidx_hbm, cidx_v)
    w_bufs = [w0_v, w1_v]
    w_sems = [sem_w0, sem_w1]
    n_q = _SPAD // _B_QSZ

    n_slots = (_NPLANES + _NW - 1) // _NW
    for slot in range(n_slots):
        j = wid + slot * _NW

        def do_job(j=j):
            n = j // _K
            psz = _PIX // 4
            plane_copies = [pltpu.async_copy(
                pred_hbm.at[j, pl.ds(i * psz, psz)],
                plane_v.at[pl.ds(i * psz, psz)], sem_p) for i in range(4)]
            w_copies = {0: pltpu.async_copy(
                w_all_hbm.at[n, :, pl.ds(0, _B_QSZ)], w_bufs[0], w_sems[0])}
            pltpu.sync_copy(wsum_hbm.at[n], wsum_v)
            with jax.named_scope("B_plane_dma"):
                for _cp in plane_copies:
                    _cp.wait()

            num_acc = jnp.zeros((_LANES,), jnp.float32)
            den_acc = jnp.zeros((_LANES,), jnp.float32)
            for q in range(n_q):
                w_buf = w_bufs[q % 2]
                with jax.named_scope("B_w_dma"):
                    w_copies[q].wait()
                if q + 1 < n_q:
                    w_copies[q + 1] = pltpu.async_copy(
                        w_all_hbm.at[n, :, pl.ds((q + 1) * _B_QSZ, _B_QSZ)],
                        w_bufs[(q + 1) % 2], w_sems[(q + 1) % 2])

                def sv_body(sv, carry, q=q, w_buf=w_buf):
                    num_c, den_c = carry
                    s_off = q * _B_QSZ + sv * _LANES
                    cvec = cidx_v[pl.ds(s_off, _LANES)]
                    pv = plsc.load_gather(plane_v, [cvec])
                    qw = jnp.zeros((_LANES,), jnp.float32)
                    for p in range(_P):
                        qv = plsc.load_gather(plane_v, [cvec + _offs[p]])
                        wv = w_buf[p, pl.ds(sv * _LANES, _LANES)]
                        qw = qw + wv * qv
                    wsv = wsum_v[pl.ds(s_off, _LANES)]
                    return num_c + qw * pv, den_c + wsv * pv

                with jax.named_scope("B_compute"):
                    num_acc, den_acc = lax.fori_loop(
                        0, _B_QSZ // _LANES, sv_body, (num_acc, den_acc))

            out_v[...] = num_acc
            pltpu.sync_copy(out_v, num_hbm.at[j])
            out_v[...] = den_acc
            pltpu.sync_copy(out_v, den_hbm.at[j])

        if (slot + 1) * _NW <= _NPLANES:
            do_job()
        else:
            @pl.when(j < _NPLANES)
            def _():
                do_job()


def _loss_body(num_ref, den_ref, out_ref):
    num = jnp.sum(num_ref[...], axis=1)
    den = jnp.sum(den_ref[...], axis=1)
    out_ref[...] = jnp.reshape(_N * _K - jnp.sum(num / den), (1, 1))


@jax.jit
def kernel(predictions, imgs):
    pred_flat = predictions.reshape(_NPLANES, _PIX)
    imgs_flat = imgs.reshape(_N * _C, _H // 8, 8, _W)
    cidx = jnp.asarray(_cidx_np)
    crow = jnp.asarray(_crow_np)
    ccol = jnp.asarray(_ccol_np)

    weights_k = pl.kernel(
        _weights_body,
        out_type=(
            jax.ShapeDtypeStruct((_N, _P, _SPAD), jnp.float32),
            jax.ShapeDtypeStruct((_N, _SPAD), jnp.float32),
        ),
        mesh=_mesh,
        scratch_types=[
            pltpu.VMEM((_BGROUPS, 8, _W), jnp.float32),
            pltpu.VMEM((_BGROUPS, 8, _W), jnp.float32),
            pltpu.VMEM((_P, _A_CSZ), jnp.float32),
            pltpu.VMEM((_P, _A_CSZ), jnp.float32),
            pltpu.VMEM((_A_CSZ,), jnp.float32),
            pltpu.VMEM((_A_CSZ,), jnp.int32),
            pltpu.VMEM((_A_CSZ,), jnp.int32),
            pltpu.SemaphoreType.DMA,
            pltpu.SemaphoreType.DMA,
        ],
        compiler_params=pltpu.CompilerParams(needs_layout_passes=False),
    )
    w_all, wsum = weights_k(imgs_flat, crow, ccol)

    numden_k = pl.kernel(
        _numden_body,
        out_type=(
            jax.ShapeDtypeStruct((_NPLANES, _LANES), jnp.float32),
            jax.ShapeDtypeStruct((_NPLANES, _LANES), jnp.float32),
        ),
        mesh=_mesh,
        scratch_types=[
            pltpu.VMEM((_PIX,), jnp.float32),
            pltpu.VMEM((_P, _B_QSZ), jnp.float32),
            pltpu.VMEM((_P, _B_QSZ), jnp.float32),
            pltpu.VMEM((_SPAD,), jnp.float32),
            pltpu.VMEM((_SPAD,), jnp.int32),
            pltpu.VMEM((_LANES,), jnp.float32),
            pltpu.SemaphoreType.DMA,
            pltpu.SemaphoreType.DMA,
            pltpu.SemaphoreType.DMA,
        ],
        compiler_params=pltpu.CompilerParams(needs_layout_passes=False),
    )
    num_p, den_p = numden_k(pred_flat, w_all, wsum, cidx)

    loss = pl.pallas_call(
        _loss_body,
        out_shape=jax.ShapeDtypeStruct((1, 1), jnp.float32),
    )(num_p, den_p)
    return loss[0, 0]


# speedup vs baseline: 1.1224x; 1.0033x over previous
import functools

import jax
import jax.numpy as jnp
import numpy as np
from jax import lax
from jax.experimental import pallas as pl
from jax.experimental.pallas import tpu as pltpu
from jax.experimental.pallas import tpu_sc as plsc

_SIGMA_I = 10.0
_SIGMA_X = 4.0
_R = 5
_SAMPLE_NUM = 1000
_H = 224
_W = 224
_N = 4
_K = 21
_C = 3
_SPAD = 1024
_PIX = _H * _W

_offs = []
_offs_ij = []
_expos = []
for _i in range(-_R, _R + 1):
    for _j in range(-_R, _R + 1):
        if _i * _i + _j * _j <= _R * _R:
            _offs.append(_i * _W + _j)
            _offs_ij.append((_i, _j))
            _expos.append(float(np.exp(-(_i * _i + _j * _j) / _SIGMA_X ** 2)))
_P = len(_offs)

_rng = np.random.default_rng(0)
_h_s = _rng.integers(_R, _H - _R, _SAMPLE_NUM)
_w_s = _rng.integers(_R, _W - _R, _SAMPLE_NUM)
_order = np.argsort(_h_s, kind="stable")
_h_sorted = _h_s[_order]
_w_sorted = _w_s[_order]
_cidx_np = np.zeros(_SPAD, np.int32)
_cidx_np[:_SAMPLE_NUM] = (_h_sorted * _W + _w_sorted).astype(np.int32)
_cidx_np[_SAMPLE_NUM:] = _cidx_np[_SAMPLE_NUM - 1]

_NC = 2
_NS = 16
_NW = _NC * _NS
_LANES = 16
_CHUNK = _SPAD // _NW

_A_CHUNKS = 8
_A_CSZ = _SPAD // _A_CHUNKS
_NPLANES = _N * _K

_h_pad = np.zeros(_SPAD, np.int64)
_h_pad[:_SAMPLE_NUM] = _h_sorted
_h_pad[_SAMPLE_NUM:] = _h_sorted[_SAMPLE_NUM - 1]
_start8 = []
_need_rows = []
for _k in range(_A_CHUNKS):
    _seg = _h_pad[_k * _A_CSZ:(_k + 1) * _A_CSZ]
    _s8 = (int(_seg.min()) - _R) & ~7
    _start8.append(_s8)
    _need_rows.append(int(_seg.max()) + _R - _s8 + 1)
_band_rows = -(-max(_need_rows) // 8) * 8
_BGROUPS = _band_rows // 8
_start8 = [min(_s, _H - _band_rows) for _s in _start8]
_h_pad_i = _h_pad.astype(np.int32)
_w_pad = np.zeros(_SPAD, np.int32)
_w_pad[:_SAMPLE_NUM] = _w_sorted
_w_pad[_SAMPLE_NUM:] = _w_sorted[_SAMPLE_NUM - 1]
_crow_np = np.zeros(_SPAD, np.int32)
for _k in range(_A_CHUNKS):
    _sl = slice(_k * _A_CSZ, (_k + 1) * _A_CSZ)
    _crow_np[_sl] = _h_pad_i[_sl] - _start8[_k]
_ccol_np = _w_pad

_mesh = plsc.VectorSubcoreMesh(
    core_axis_name="c", subcore_axis_name="s", num_cores=_NC, num_subcores=_NS
)


def _weights_body(imgs_hbm, crow_hbm, ccol_hbm, w_hbm, wsum_hbm, plane0_v,
                  plane1_v, acc2_v, w_v, wsum_v, crow_v, ccol_v, sem0, sem1):
    wid = lax.axis_index("s") * _NC + lax.axis_index("c")
    n = wid // _A_CHUNKS
    chunk = wid % _A_CHUNKS
    base = chunk * _A_CSZ

    pltpu.sync_copy(crow_hbm.at[pl.ds(base, _A_CSZ)], crow_v)
    pltpu.sync_copy(ccol_hbm.at[pl.ds(base, _A_CSZ)], ccol_v)

    neg_inv_s2 = -1.0 / (_SIGMA_I ** 2)
    planes = [plane0_v, plane1_v]
    sems = [sem0, sem1]

    bgrp = jnp.int32(_start8[0] // 8)
    for _k in range(1, _A_CHUNKS):
        bgrp = jnp.where(chunk == _k, jnp.int32(_start8[_k] // 8), bgrp)

    def start_plane(c, bi):
        return [pltpu.async_copy(
            imgs_hbm.at[n * _C + c, pl.ds(bgrp, _BGROUPS)],
            planes[bi], sems[bi])]

    copies = {0: start_plane(0, 0), 1: start_plane(1, 1)}
    for c in range(_C):
        buf = planes[c % 2]
        for _cp in copies[c]:
            _cp.wait()

        def sv_body(sv, _, c=c, buf=buf):
            rvec = crow_v[pl.ds(sv * _LANES, _LANES)]
            kvec = ccol_v[pl.ds(sv * _LANES, _LANES)]
            centv = plsc.load_gather(buf, [rvec >> 3, rvec & 7, kvec])
            for p in range(_P):
                oi, oj = _offs_ij[p]
                rr = rvec + oi
                gv = plsc.load_gather(buf, [rr >> 3, rr & 7, kvec + oj])
                d = gv - centv
                if c == 0:
                    w_slc = acc2_v.at[p, pl.ds(sv * _LANES, _LANES)]
                    w_slc[...] = d * d
                else:
                    plsc.addupdate(acc2_v.at[p, pl.ds(sv * _LANES, _LANES)], d * d)
            return _

        lax.fori_loop(0, _A_CSZ // _LANES, sv_body, None)
        if c + 2 < _C:
            copies[c + 2] = start_plane(c + 2, c % 2)

    def exp_body(sv, _):
        s_base = base + sv * _LANES
        lanes = lax.iota(jnp.int32, _LANES) + s_base
        valid = lanes < _SAMPLE_NUM
        wsums = [jnp.zeros((_LANES,), jnp.float32) for _i in range(3)]
        for p in range(_P):
            a = acc2_v[p, pl.ds(sv * _LANES, _LANES)]
            wv = jnp.exp(a * neg_inv_s2) * _expos[p]
            wv = jnp.where(valid, wv, 0.0)
            w_v[p, pl.ds(sv * _LANES, _LANES)] = wv
            wsums[p % 3] = wsums[p % 3] + wv
        wsum_v[pl.ds(sv * _LANES, _LANES)] = wsums[0] + (wsums[1] + wsums[2])
        return _

    lax.fori_loop(0, _A_CSZ // _LANES, exp_body, None)

    pltpu.sync_copy(w_v, w_hbm.at[n, :, pl.ds(base, _A_CSZ)])
    pltpu.sync_copy(wsum_v, wsum_hbm.at[n, pl.ds(base, _A_CSZ)])


_B_QSZ = 256


def _numden_body(pred_hbm, w_all_hbm, wsum_hbm, cidx_hbm, num_hbm, den_hbm,
                 plane_v, w0_v, w1_v, wsum_v, cidx_v, out_v, sem_p, sem_w0,
                 sem_w1):
    wid = lax.axis_index("s") * _NC + lax.axis_index("c")

    pltpu.sync_copy(cidx_hbm, cidx_v)
    w_bufs = [w0_v, w1_v]
    w_sems = [sem_w0, sem_w1]
    n_q = _SPAD // _B_QSZ

    n_slots = (_NPLANES + _NW - 1) // _NW
    for slot in range(n_slots):
        j = wid + slot * _NW

        def do_job(j=j):
            n = j // _K
            psz = _PIX // 4
            plane_copies = [pltpu.async_copy(
                pred_hbm.at[j, pl.ds(i * psz, psz)],
                plane_v.at[pl.ds(i * psz, psz)], sem_p) for i in range(4)]
            w_copies = {0: pltpu.async_copy(
                w_all_hbm.at[n, :, pl.ds(0, _B_QSZ)], w_bufs[0], w_sems[0])}
            pltpu.sync_copy(wsum_hbm.at[n], wsum_v)
            for _cp in plane_copies:
                _cp.wait()

            num_acc = jnp.zeros((_LANES,), jnp.float32)
            den_acc = jnp.zeros((_LANES,), jnp.float32)
            for q in range(n_q):
                w_buf = w_bufs[q % 2]
                w_copies[q].wait()
                if q + 1 < n_q:
                    w_copies[q + 1] = pltpu.async_copy(
                        w_all_hbm.at[n, :, pl.ds((q + 1) * _B_QSZ, _B_QSZ)],
                        w_bufs[(q + 1) % 2], w_sems[(q + 1) % 2])

                def sv_body(sv, carry, q=q, w_buf=w_buf):
                    num_c, den_c = carry
                    s_off = q * _B_QSZ + sv * _LANES
                    cvec = cidx_v[pl.ds(s_off, _LANES)]
                    pv = plsc.load_gather(plane_v, [cvec])
                    qw = jnp.zeros((_LANES,), jnp.float32)
                    for p in range(_P):
                        qv = plsc.load_gather(plane_v, [cvec + _offs[p]])
                        wv = w_buf[p, pl.ds(sv * _LANES, _LANES)]
                        qw = qw + wv * qv
                    wsv = wsum_v[pl.ds(s_off, _LANES)]
                    return num_c + qw * pv, den_c + wsv * pv

                num_acc, den_acc = lax.fori_loop(
                    0, _B_QSZ // _LANES, sv_body, (num_acc, den_acc))

            out_v[...] = num_acc
            pltpu.sync_copy(out_v, num_hbm.at[j])
            out_v[...] = den_acc
            pltpu.sync_copy(out_v, den_hbm.at[j])

        if (slot + 1) * _NW <= _NPLANES:
            do_job()
        else:
            @pl.when(j < _NPLANES)
            def _():
                do_job()


def _loss_body(num_ref, den_ref, out_ref):
    num = jnp.sum(num_ref[...], axis=1)
    den = jnp.sum(den_ref[...], axis=1)
    out_ref[...] = jnp.reshape(_N * _K - jnp.sum(num / den), (1, 1))


@jax.jit
def kernel(predictions, imgs):
    pred_flat = predictions.reshape(_NPLANES, _PIX)
    imgs_flat = imgs.reshape(_N * _C, _H // 8, 8, _W)
    cidx = jnp.asarray(_cidx_np)
    crow = jnp.asarray(_crow_np)
    ccol = jnp.asarray(_ccol_np)

    weights_k = pl.kernel(
        _weights_body,
        out_type=(
            jax.ShapeDtypeStruct((_N, _P, _SPAD), jnp.float32),
            jax.ShapeDtypeStruct((_N, _SPAD), jnp.float32),
        ),
        mesh=_mesh,
        scratch_types=[
            pltpu.VMEM((_BGROUPS, 8, _W), jnp.float32),
            pltpu.VMEM((_BGROUPS, 8, _W), jnp.float32),
            pltpu.VMEM((_P, _A_CSZ), jnp.float32),
            pltpu.VMEM((_P, _A_CSZ), jnp.float32),
            pltpu.VMEM((_A_CSZ,), jnp.float32),
            pltpu.VMEM((_A_CSZ,), jnp.int32),
            pltpu.VMEM((_A_CSZ,), jnp.int32),
            pltpu.SemaphoreType.DMA,
            pltpu.SemaphoreType.DMA,
        ],
        compiler_params=pltpu.CompilerParams(needs_layout_passes=False),
    )
    w_all, wsum = weights_k(imgs_flat, crow, ccol)

    numden_k = pl.kernel(
        _numden_body,
        out_type=(
            jax.ShapeDtypeStruct((_NPLANES, _LANES), jnp.float32),
            jax.ShapeDtypeStruct((_NPLANES, _LANES), jnp.float32),
        ),
        mesh=_mesh,
        scratch_types=[
            pltpu.VMEM((_PIX,), jnp.float32),
            pltpu.VMEM((_P, _B_QSZ), jnp.float32),
            pltpu.VMEM((_P, _B_QSZ), jnp.float32),
            pltpu.VMEM((_SPAD,), jnp.float32),
            pltpu.VMEM((_SPAD,), jnp.int32),
            pltpu.VMEM((_LANES,), jnp.float32),
            pltpu.SemaphoreType.DMA,
            pltpu.SemaphoreType.DMA,
            pltpu.SemaphoreType.DMA,
        ],
        compiler_params=pltpu.CompilerParams(needs_layout_passes=False),
    )
    num_p, den_p = numden_k(pred_flat, w_all, wsum, cidx)

    loss = pl.pallas_call(
        _loss_body,
        out_shape=jax.ShapeDtypeStruct((1, 1), jnp.float32),
    )(num_p, den_p)
    return loss[0, 0]


# speedup vs baseline: 1.1236x; 1.0011x over previous
import functools

import jax
import jax.numpy as jnp
import numpy as np
from jax import lax
from jax.experimental import pallas as pl
from jax.experimental.pallas import tpu as pltpu
from jax.experimental.pallas import tpu_sc as plsc

_SIGMA_I = 10.0
_SIGMA_X = 4.0
_R = 5
_SAMPLE_NUM = 1000
_H = 224
_W = 224
_N = 4
_K = 21
_C = 3
_SPAD = 1024
_PIX = _H * _W

_offs = []
_offs_ij = []
_expos = []
for _i in range(-_R, _R + 1):
    for _j in range(-_R, _R + 1):
        if _i * _i + _j * _j <= _R * _R:
            _offs.append(_i * _W + _j)
            _offs_ij.append((_i, _j))
            _expos.append(float(np.exp(-(_i * _i + _j * _j) / _SIGMA_X ** 2)))
_P = len(_offs)

_rng = np.random.default_rng(0)
_h_s = _rng.integers(_R, _H - _R, _SAMPLE_NUM)
_w_s = _rng.integers(_R, _W - _R, _SAMPLE_NUM)
_order = np.argsort(_h_s, kind="stable")
_h_sorted = _h_s[_order]
_w_sorted = _w_s[_order]
_cidx_np = np.zeros(_SPAD, np.int32)
_cidx_np[:_SAMPLE_NUM] = (_h_sorted * _W + _w_sorted).astype(np.int32)
_cidx_np[_SAMPLE_NUM:] = _cidx_np[_SAMPLE_NUM - 1]

_NC = 2
_NS = 16
_NW = _NC * _NS
_LANES = 16

_A_CHUNKS = 8
_A_CSZ = _SPAD // _A_CHUNKS
_NPLANES = _N * _K

_h_pad = np.zeros(_SPAD, np.int64)
_h_pad[:_SAMPLE_NUM] = _h_sorted
_h_pad[_SAMPLE_NUM:] = _h_sorted[_SAMPLE_NUM - 1]
_start8 = []
_need_rows = []
for _k in range(_A_CHUNKS):
    _seg = _h_pad[_k * _A_CSZ:(_k + 1) * _A_CSZ]
    _s8 = (int(_seg.min()) - _R) & ~7
    _start8.append(_s8)
    _need_rows.append(int(_seg.max()) + _R - _s8 + 1)
_band_rows = -(-max(_need_rows) // 8) * 8
_BGROUPS = _band_rows // 8
_start8 = [min(_s, _H - _band_rows) for _s in _start8]
_h_pad_i = _h_pad.astype(np.int32)
_w_pad = np.zeros(_SPAD, np.int32)
_w_pad[:_SAMPLE_NUM] = _w_sorted
_w_pad[_SAMPLE_NUM:] = _w_sorted[_SAMPLE_NUM - 1]
_crow_np = np.zeros(_SPAD, np.int32)
for _k in range(_A_CHUNKS):
    _sl = slice(_k * _A_CSZ, (_k + 1) * _A_CSZ)
    _crow_np[_sl] = _h_pad_i[_sl] - _start8[_k]
_ccol_np = _w_pad

_mesh = plsc.VectorSubcoreMesh(
    core_axis_name="c", subcore_axis_name="s", num_cores=_NC, num_subcores=_NS
)


def _weights_body(imgs_hbm, crow_hbm, ccol_hbm, w_hbm, wsum_hbm, plane0_v,
                  plane1_v, acc2_v, w_v, wsum_v, crow_v, ccol_v, sem0, sem1):
    wid = lax.axis_index("s") * _NC + lax.axis_index("c")
    n = wid // _A_CHUNKS
    chunk = wid % _A_CHUNKS
    base = chunk * _A_CSZ

    pltpu.sync_copy(crow_hbm.at[pl.ds(base, _A_CSZ)], crow_v)
    pltpu.sync_copy(ccol_hbm.at[pl.ds(base, _A_CSZ)], ccol_v)

    neg_inv_s2 = -1.0 / (_SIGMA_I ** 2)
    planes = [plane0_v, plane1_v]
    sems = [sem0, sem1]

    bgrp = jnp.int32(_start8[0] // 8)
    for _k in range(1, _A_CHUNKS):
        bgrp = jnp.where(chunk == _k, jnp.int32(_start8[_k] // 8), bgrp)

    def start_plane(c, bi):
        return [pltpu.async_copy(
            imgs_hbm.at[n * _C + c, pl.ds(bgrp, _BGROUPS)],
            planes[bi], sems[bi])]

    copies = {0: start_plane(0, 0), 1: start_plane(1, 1)}
    for c in range(_C):
        buf = planes[c % 2]
        for _cp in copies[c]:
            _cp.wait()

        def sv_body(sv, _, c=c, buf=buf):
            rvec = crow_v[pl.ds(sv * _LANES, _LANES)]
            kvec = ccol_v[pl.ds(sv * _LANES, _LANES)]
            centv = plsc.load_gather(buf, [rvec >> 3, rvec & 7, kvec])
            for p in range(_P):
                oi, oj = _offs_ij[p]
                rr = rvec + oi
                gv = plsc.load_gather(buf, [rr >> 3, rr & 7, kvec + oj])
                d = gv - centv
                if c == 0:
                    w_slc = acc2_v.at[p, pl.ds(sv * _LANES, _LANES)]
                    w_slc[...] = d * d
                else:
                    plsc.addupdate(acc2_v.at[p, pl.ds(sv * _LANES, _LANES)], d * d)
            return _

        lax.fori_loop(0, _A_CSZ // _LANES, sv_body, None)
        if c + 2 < _C:
            copies[c + 2] = start_plane(c + 2, c % 2)

    def exp_body(sv, _):
        s_base = base + sv * _LANES
        lanes = lax.iota(jnp.int32, _LANES) + s_base
        valid = lanes < _SAMPLE_NUM
        wsums = [jnp.zeros((_LANES,), jnp.float32) for _i in range(3)]
        for p in range(_P):
            a = acc2_v[p, pl.ds(sv * _LANES, _LANES)]
            wv = jnp.exp(a * neg_inv_s2) * _expos[p]
            wv = jnp.where(valid, wv, 0.0)
            w_v[p, pl.ds(sv * _LANES, _LANES)] = wv
            wsums[p % 3] = wsums[p % 3] + wv
        wsum_v[pl.ds(sv * _LANES, _LANES)] = wsums[0] + (wsums[1] + wsums[2])
        return _

    lax.fori_loop(0, _A_CSZ // _LANES, exp_body, None)

    pltpu.sync_copy(w_v, w_hbm.at[n, :, pl.ds(base, _A_CSZ)])
    pltpu.sync_copy(wsum_v, wsum_hbm.at[n, pl.ds(base, _A_CSZ)])


_B_QSZ = 256


def _numden_body(pred_hbm, w_all_hbm, wsum_hbm, cidx_hbm, num_hbm, den_hbm,
                 plane_v, w0_v, w1_v, wsum_v, cidx_v, out_v, sem_p, sem_w0,
                 sem_w1):
    wid = lax.axis_index("s") * _NC + lax.axis_index("c")

    pltpu.sync_copy(cidx_hbm, cidx_v)
    w_bufs = [w0_v, w1_v]
    w_sems = [sem_w0, sem_w1]
    n_q = _SPAD // _B_QSZ

    n_slots = (_NPLANES + _NW - 1) // _NW
    for slot in range(n_slots):
        j = wid + slot * _NW

        def do_job(j=j):
            n = j // _K
            psz = _PIX // 4
            plane_copies = [pltpu.async_copy(
                pred_hbm.at[j, pl.ds(i * psz, psz)],
                plane_v.at[pl.ds(i * psz, psz)], sem_p) for i in range(4)]
            w_copies = {0: pltpu.async_copy(
                w_all_hbm.at[n, :, pl.ds(0, _B_QSZ)], w_bufs[0], w_sems[0])}
            pltpu.sync_copy(wsum_hbm.at[n], wsum_v)
            for _cp in plane_copies:
                _cp.wait()

            num_acc = jnp.zeros((_LANES,), jnp.float32)
            den_acc = jnp.zeros((_LANES,), jnp.float32)
            for q in range(n_q):
                w_buf = w_bufs[q % 2]
                w_copies[q].wait()
                if q + 1 < n_q:
                    w_copies[q + 1] = pltpu.async_copy(
                        w_all_hbm.at[n, :, pl.ds((q + 1) * _B_QSZ, _B_QSZ)],
                        w_bufs[(q + 1) % 2], w_sems[(q + 1) % 2])

                def sv_body(sv, carry, q=q, w_buf=w_buf):
                    num_c, den_c = carry
                    s_off = q * _B_QSZ + sv * _LANES
                    cvec = cidx_v[pl.ds(s_off, _LANES)]
                    pv = plsc.load_gather(plane_v, [cvec])
                    qw = jnp.zeros((_LANES,), jnp.float32)
                    for p in range(_P):
                        qv = plsc.load_gather(plane_v, [cvec + _offs[p]])
                        wv = w_buf[p, pl.ds(sv * _LANES, _LANES)]
                        qw = qw + wv * qv
                    wsv = wsum_v[pl.ds(s_off, _LANES)]
                    return num_c + qw * pv, den_c + wsv * pv

                num_acc, den_acc = lax.fori_loop(
                    0, _B_QSZ // _LANES, sv_body, (num_acc, den_acc))

            out_v[...] = num_acc
            pltpu.sync_copy(out_v, num_hbm.at[j])
            out_v[...] = den_acc
            pltpu.sync_copy(out_v, den_hbm.at[j])

        if (slot + 1) * _NW <= _NPLANES:
            do_job()
        else:
            @pl.when(j < _NPLANES)
            def _():
                do_job()


def _loss_body(num_ref, den_ref, out_ref):
    num = jnp.sum(num_ref[...], axis=1)
    den = jnp.sum(den_ref[...], axis=1)
    out_ref[...] = jnp.reshape(_N * _K - jnp.sum(num / den), (1, 1))


@jax.jit
def kernel(predictions, imgs):
    pred_flat = predictions.reshape(_NPLANES, _PIX)
    imgs_flat = imgs.reshape(_N * _C, _H // 8, 8, _W)
    cidx = jnp.asarray(_cidx_np)
    crow = jnp.asarray(_crow_np)
    ccol = jnp.asarray(_ccol_np)

    weights_k = pl.kernel(
        _weights_body,
        out_type=(
            jax.ShapeDtypeStruct((_N, _P, _SPAD), jnp.float32),
            jax.ShapeDtypeStruct((_N, _SPAD), jnp.float32),
        ),
        mesh=_mesh,
        scratch_types=[
            pltpu.VMEM((_BGROUPS, 8, _W), jnp.float32),
            pltpu.VMEM((_BGROUPS, 8, _W), jnp.float32),
            pltpu.VMEM((_P, _A_CSZ), jnp.float32),
            pltpu.VMEM((_P, _A_CSZ), jnp.float32),
            pltpu.VMEM((_A_CSZ,), jnp.float32),
            pltpu.VMEM((_A_CSZ,), jnp.int32),
            pltpu.VMEM((_A_CSZ,), jnp.int32),
            pltpu.SemaphoreType.DMA,
            pltpu.SemaphoreType.DMA,
        ],
        compiler_params=pltpu.CompilerParams(needs_layout_passes=False),
    )
    w_all, wsum = weights_k(imgs_flat, crow, ccol)

    numden_k = pl.kernel(
        _numden_body,
        out_type=(
            jax.ShapeDtypeStruct((_NPLANES, _LANES), jnp.float32),
            jax.ShapeDtypeStruct((_NPLANES, _LANES), jnp.float32),
        ),
        mesh=_mesh,
        scratch_types=[
            pltpu.VMEM((_PIX,), jnp.float32),
            pltpu.VMEM((_P, _B_QSZ), jnp.float32),
            pltpu.VMEM((_P, _B_QSZ), jnp.float32),
            pltpu.VMEM((_SPAD,), jnp.float32),
            pltpu.VMEM((_SPAD,), jnp.int32),
            pltpu.VMEM((_LANES,), jnp.float32),
            pltpu.SemaphoreType.DMA,
            pltpu.SemaphoreType.DMA,
            pltpu.SemaphoreType.DMA,
        ],
        compiler_params=pltpu.CompilerParams(needs_layout_passes=False),
    )
    num_p, den_p = numden_k(pred_flat, w_all, wsum, cidx)

    loss = pl.pallas_call(
        _loss_body,
        out_shape=jax.ShapeDtypeStruct((1, 1), jnp.float32),
    )(num_p, den_p)
    return loss[0, 0]
